# baseline (speedup 1.0000x reference)
import functools

import jax
import jax.numpy as jnp
from jax import lax
from jax.experimental import pallas as pl
from jax.experimental.pallas import tpu as pltpu
from jax.experimental.pallas import tpu_sc as plsc

_REG = 0.001
_B = 16384
_D = 64
_NC = 2
_NS = 16
_L = 16
_NW = _NC * _NS
_BPW = _B // _NW
_CHUNK = 128
_NCHUNK = _BPW // _CHUNK

_sc_mesh = plsc.VectorSubcoreMesh(
    core_axis_name="c", subcore_axis_name="s",
    num_cores=_NC, num_subcores=_NS)
_sc_params = pltpu.CompilerParams(
    needs_layout_passes=False, use_tc_tiling_on_sc=False)


def _bf16_pair(ref, r, cc):
    w = plsc.bitcast(ref[r, pl.ds(cc * 2 * _L, 2 * _L)], jnp.int32)
    lo = plsc.bitcast(lax.shift_left(w, 16), jnp.float32)
    hi = plsc.bitcast(
        lax.bitwise_and(w, jnp.int32(-65536)), jnp.float32)
    return lo, hi


def _shuffled_row(v):
    return lax.bitwise_or(
        lax.bitwise_or(
            lax.shift_left(lax.shift_right_logical(v, 4), 4),
            lax.shift_left(lax.bitwise_and(v, 7), 1)),
        lax.bitwise_and(lax.shift_right_logical(v, 3), 1))


def _sc_item_body(ii_hbm, ij_hbm, et_hbm,
                  svi_hbm, svj_hbm, reg_hbm,
                  iidx_v, jidx_v, ipr_v, jpr_v,
                  via, vja, vib, vjb, reg_v,
                  sem_a, sem_b, sem_w):
    c = lax.axis_index("c")
    s = lax.axis_index("s")
    wid = s * _NC + c
    base = wid * _BPW

    pltpu.sync_copy(ii_hbm.at[pl.ds(base, _BPW)], iidx_v)
    pltpu.sync_copy(ij_hbm.at[pl.ds(base, _BPW)], jidx_v)

    def prep(i, _):
        sl = pl.ds(i * _L, _L)
        ipr_v[sl] = _shuffled_row(iidx_v[sl])
        jpr_v[sl] = _shuffled_row(jidx_v[sl])
        return 0
    lax.fori_loop(0, _BPW // _L, prep, 0)

    slots = ((via, vja, sem_a), (vib, vjb, sem_b))

    def fire(k, slot):
        vi_v, vj_v, sem = slot
        sl = pl.ds(k * _CHUNK, _CHUNK)
        return (pltpu.async_copy(et_hbm.at[ipr_v.at[sl]], vi_v, sem),
                pltpu.async_copy(et_hbm.at[jpr_v.at[sl]], vj_v, sem))

    reg_acc = jnp.zeros((_L,), jnp.float32)
    inflight = fire(0, slots[0])
    writes = ()
    for k in range(_NCHUNK):
        vi_v, vj_v, _ = slots[k % 2]
        for cp in inflight:
            cp.wait()
        if k + 1 < _NCHUNK:
            inflight = fire(k + 1, slots[(k + 1) % 2])

        def sq(g, reg_acc, vi_v=vi_v, vj_v=vj_v):
            for l in range(_L):
                r = g * _L + l
                for cc in range(_D // (2 * _L)):
                    vlo, vhi = _bf16_pair(vi_v, r, cc)
                    wlo, whi = _bf16_pair(vj_v, r, cc)
                    reg_acc = (reg_acc + vlo * vlo + vhi * vhi
                               + wlo * wlo + whi * whi)
            return reg_acc

        reg_acc = lax.fori_loop(0, _CHUNK // _L, sq, reg_acc)
        for cp in writes:
            cp.wait()
        dst = pl.ds(base + k * _CHUNK, _CHUNK)
        writes = (pltpu.async_copy(vi_v, svi_hbm.at[dst], sem_w),
                  pltpu.async_copy(vj_v, svj_hbm.at[dst], sem_w))
    for cp in writes:
        cp.wait()

    reg_v[...] = reg_acc
    pltpu.sync_copy(reg_v, reg_hbm.at[pl.ds(wid * _L, _L)])


_sc_item = functools.partial(
    pl.kernel,
    out_type=(
        jax.ShapeDtypeStruct((_B, _D), jnp.bfloat16),
        jax.ShapeDtypeStruct((_B, _D), jnp.bfloat16),
        jax.ShapeDtypeStruct((_NW * _L,), jnp.float32),
    ),
    mesh=_sc_mesh,
    compiler_params=_sc_params,
    scratch_types=[
        pltpu.VMEM((_BPW,), jnp.int32),
        pltpu.VMEM((_BPW,), jnp.int32),
        pltpu.VMEM((_BPW,), jnp.int32),
        pltpu.VMEM((_BPW,), jnp.int32),
        pltpu.VMEM((_CHUNK, _D), jnp.bfloat16),
        pltpu.VMEM((_CHUNK, _D), jnp.bfloat16),
        pltpu.VMEM((_CHUNK, _D), jnp.bfloat16),
        pltpu.VMEM((_CHUNK, _D), jnp.bfloat16),
        pltpu.VMEM((_L,), jnp.float32),
        pltpu.SemaphoreType.DMA,
        pltpu.SemaphoreType.DMA,
        pltpu.SemaphoreType.DMA,
    ],
)(_sc_item_body)


def _sc_user_body(user_hbm, eut_hbm, svi_hbm, svj_hbm,
                  pi_hbm, pj_hbm, reg_hbm,
                  uidx_v, upr_v,
                  ua, via, vja, ub, vib, vjb, pi_v, pj_v, reg_v,
                  sem_a, sem_b):
    c = lax.axis_index("c")
    s = lax.axis_index("s")
    wid = s * _NC + c
    base = wid * _BPW

    pltpu.sync_copy(user_hbm.at[pl.ds(base, _BPW)], uidx_v)

    def prep(i, _):
        sl = pl.ds(i * _L, _L)
        upr_v[sl] = _shuffled_row(uidx_v[sl])
        return 0
    lax.fori_loop(0, _BPW // _L, prep, 0)

    slots = ((ua, via, vja, sem_a), (ub, vib, vjb, sem_b))

    def fire(k, slot):
        u_v, vi_v, vj_v, sem = slot
        sl = pl.ds(k * _CHUNK, _CHUNK)
        ssl = pl.ds(base + k * _CHUNK, _CHUNK)
        return (pltpu.async_copy(eut_hbm.at[upr_v.at[sl]], u_v, sem),
                pltpu.async_copy(svi_hbm.at[ssl], vi_v, sem),
                pltpu.async_copy(svj_hbm.at[ssl], vj_v, sem))

    lane = lax.iota(jnp.int32, _L)
    reg_acc = jnp.zeros((_L,), jnp.float32)
    inflight = fire(0, slots[0])
    for k in range(_NCHUNK):
        u_v, vi_v, vj_v, _ = slots[k % 2]
        for cp in inflight:
            cp.wait()
        if k + 1 < _NCHUNK:
            inflight = fire(k + 1, slots[(k + 1) % 2])

        def group(g, reg_acc, u_v=u_v, vi_v=vi_v, vj_v=vj_v, k=k):
            acc_i = jnp.zeros((_L,), jnp.float32)
            acc_j = jnp.zeros((_L,), jnp.float32)
            for l in range(_L):
                r = g * _L + l
                ss_i = jnp.zeros((_L,), jnp.float32)
                ss_j = jnp.zeros((_L,), jnp.float32)
                for cc in range(_D // (2 * _L)):
                    ulo, uhi = _bf16_pair(u_v, r, cc)
                    vlo, vhi = _bf16_pair(vi_v, r, cc)
                    wlo, whi = _bf16_pair(vj_v, r, cc)
                    ss_i = ss_i + ulo * vlo + uhi * vhi
                    ss_j = ss_j + ulo * wlo + uhi * whi
                    reg_acc = reg_acc + ulo * ulo + uhi * uhi
                acc_i = jnp.where(lane == l, jnp.sum(ss_i), acc_i)
                acc_j = jnp.where(lane == l, jnp.sum(ss_j), acc_j)
            row = pl.ds(k * _CHUNK + g * _L, _L)
            pi_v[row] = acc_i
            pj_v[row] = acc_j
            return reg_acc

        reg_acc = lax.fori_loop(0, _CHUNK // _L, group, reg_acc)

    reg_v[...] = reg_acc
    pltpu.sync_copy(pi_v, pi_hbm.at[pl.ds(base, _BPW)])
    pltpu.sync_copy(pj_v, pj_hbm.at[pl.ds(base, _BPW)])
    pltpu.sync_copy(reg_v, reg_hbm.at[pl.ds(wid * _L, _L)])


_sc_user = functools.partial(
    pl.kernel,
    out_type=(
        jax.ShapeDtypeStruct((_B,), jnp.float32),
        jax.ShapeDtypeStruct((_B,), jnp.float32),
        jax.ShapeDtypeStruct((_NW * _L,), jnp.float32),
    ),
    mesh=_sc_mesh,
    compiler_params=_sc_params,
    scratch_types=[
        pltpu.VMEM((_BPW,), jnp.int32),
        pltpu.VMEM((_BPW,), jnp.int32),
        pltpu.VMEM((_CHUNK, _D), jnp.bfloat16),
        pltpu.VMEM((_CHUNK, _D), jnp.bfloat16),
        pltpu.VMEM((_CHUNK, _D), jnp.bfloat16),
        pltpu.VMEM((_CHUNK, _D), jnp.bfloat16),
        pltpu.VMEM((_CHUNK, _D), jnp.bfloat16),
        pltpu.VMEM((_CHUNK, _D), jnp.bfloat16),
        pltpu.VMEM((_BPW,), jnp.float32),
        pltpu.VMEM((_BPW,), jnp.float32),
        pltpu.VMEM((_L,), jnp.float32),
        pltpu.SemaphoreType.DMA,
        pltpu.SemaphoreType.DMA,
    ],
)(_sc_user_body)


def _pair_body(t_ref, o_ref):
    x = jnp.swapaxes(t_ref[...], 0, 1)
    q = x.shape[0] // 16
    x4 = x.reshape(q, 2, 8, _D)
    out = jnp.concatenate([x4[:, 0], x4[:, 1]], axis=-1)
    o_ref[...] = out.reshape(q * 8 * 2 * _D).astype(jnp.bfloat16)


def _linear_rows(table):
    n = table.shape[0]
    m = 8 * ((n + 15) // 16)
    blk = 8192
    grid = (m + blk - 1) // blk
    flat = pl.pallas_call(
        _pair_body,
        grid=(grid,),
        in_specs=[pl.BlockSpec((_D, 2 * blk), lambda i: (0, i))],
        out_specs=pl.BlockSpec((blk * 2 * _D,), lambda i: (i,)),
        out_shape=jax.ShapeDtypeStruct((m * 2 * _D,), jnp.bfloat16),
    )(table.T)
    return flat.reshape(2 * m, _D)


def _loss_body(pi_ref, pj_ref, regi_ref, regu_ref, out_ref):
    x = pi_ref[...] - pj_ref[...]
    ls = jnp.minimum(x, 0.0) - jnp.log(1.0 + jnp.exp(-jnp.abs(x)))
    reg = jnp.sum(regi_ref[...]) + jnp.sum(regu_ref[...])
    out_ref[0, 0] = _REG * reg - jnp.sum(ls)


_loss_call = pl.pallas_call(
    _loss_body,
    out_shape=jax.ShapeDtypeStruct((1, 1), jnp.float32),
    out_specs=pl.BlockSpec(memory_space=pltpu.SMEM),
)


def kernel(user, item_i, item_j, embed_user, embed_item):
    ei2 = _linear_rows(embed_item)
    svi, svj, regij = _sc_item(item_i, item_j, ei2)
    eu2 = _linear_rows(embed_user)
    pi, pj, regu = _sc_user(user, eu2, svi, svj)
    loss = _loss_call(pi.reshape(_B // 128, 128),
                      pj.reshape(_B // 128, 128),
                      regij.reshape(_NW * _L // 128, 128),
                      regu.reshape(_NW * _L // 128, 128))[0, 0]
    return (pi, pj, loss)


# speedup vs baseline: 2.1139x; 2.1139x over previous
import functools

import jax
import jax.numpy as jnp
from jax import lax
from jax.experimental import pallas as pl
from jax.experimental.pallas import tpu as pltpu
from jax.experimental.pallas import tpu_sc as plsc

_REG = 0.001
_B = 16384
_D = 64
_NC = 2
_NS = 16
_L = 16
_NW = _NC * _NS
_BPW = _B // _NW
_CHUNK = 128
_NCHUNK = _BPW // _CHUNK

_sc_mesh = plsc.VectorSubcoreMesh(
    core_axis_name="c", subcore_axis_name="s",
    num_cores=_NC, num_subcores=_NS)
_sc_params = pltpu.CompilerParams(
    needs_layout_passes=False, use_tc_tiling_on_sc=False)


def _bf16_pair(ref, r, cc):
    w = ref[r, pl.ds(cc * _L, _L)]
    lo = plsc.bitcast(lax.shift_left(w, 16), jnp.float32)
    hi = plsc.bitcast(
        lax.bitwise_and(w, jnp.int32(-65536)), jnp.float32)
    return lo, hi


def _shuffled_row(v):
    return lax.bitwise_or(
        lax.bitwise_or(
            lax.shift_left(lax.shift_right_logical(v, 5), 5),
            lax.shift_left(lax.bitwise_and(v, 7), 2)),
        lax.bitwise_and(lax.shift_right_logical(v, 3), 3))


def _sc_item_body(ii_hbm, ij_hbm, et_hbm,
                  svi_hbm, svj_hbm, reg_hbm,
                  iidx_v, jidx_v, ipr_v, jpr_v,
                  via, vja, vib, vjb, reg_v,
                  sem_a, sem_b, sem_w):
    c = lax.axis_index("c")
    s = lax.axis_index("s")
    wid = s * _NC + c
    base = wid * _BPW

    pltpu.sync_copy(ii_hbm.at[pl.ds(base, _BPW)], iidx_v)
    pltpu.sync_copy(ij_hbm.at[pl.ds(base, _BPW)], jidx_v)

    def prep(i, _):
        sl = pl.ds(i * _L, _L)
        ipr_v[sl] = _shuffled_row(iidx_v[sl])
        jpr_v[sl] = _shuffled_row(jidx_v[sl])
        return 0
    lax.fori_loop(0, _BPW // _L, prep, 0)

    slots = ((via, vja, sem_a), (vib, vjb, sem_b))

    def fire(k, slot):
        vi_v, vj_v, sem = slot
        sl = pl.ds(k * _CHUNK, _CHUNK)
        return (pltpu.async_copy(et_hbm.at[ipr_v.at[sl]], vi_v, sem),
                pltpu.async_copy(et_hbm.at[jpr_v.at[sl]], vj_v, sem))

    reg_acc = jnp.zeros((_L,), jnp.float32)
    inflight = fire(0, slots[0])
    writes = ()
    for k in range(_NCHUNK):
        vi_v, vj_v, _ = slots[k % 2]
        for cp in inflight:
            cp.wait()
        if k + 1 < _NCHUNK:
            inflight = fire(k + 1, slots[(k + 1) % 2])

        def sq(g, reg_acc, vi_v=vi_v, vj_v=vj_v):
            for l in range(_L):
                r = g * _L + l
                for cc in range(2):
                    vlo, vhi = _bf16_pair(vi_v, r, cc)
                    wlo, whi = _bf16_pair(vj_v, r, cc)
                    reg_acc = (reg_acc + vlo * vlo + vhi * vhi
                               + wlo * wlo + whi * whi)
            return reg_acc

        reg_acc = lax.fori_loop(0, _CHUNK // _L, sq, reg_acc)
        for cp in writes:
            cp.wait()
        dst = pl.ds(base + k * _CHUNK, _CHUNK)
        writes = (pltpu.async_copy(vi_v, svi_hbm.at[dst], sem_w),
                  pltpu.async_copy(vj_v, svj_hbm.at[dst], sem_w))
    for cp in writes:
        cp.wait()

    reg_v[...] = reg_acc
    pltpu.sync_copy(reg_v, reg_hbm.at[pl.ds(wid * _L, _L)])


_sc_item = functools.partial(
    pl.kernel,
    out_type=(
        jax.ShapeDtypeStruct((_B, _D // 2), jnp.int32),
        jax.ShapeDtypeStruct((_B, _D // 2), jnp.int32),
        jax.ShapeDtypeStruct((_NW * _L,), jnp.float32),
    ),
    mesh=_sc_mesh,
    compiler_params=_sc_params,
    scratch_types=[
        pltpu.VMEM((_BPW,), jnp.int32),
        pltpu.VMEM((_BPW,), jnp.int32),
        pltpu.VMEM((_BPW,), jnp.int32),
        pltpu.VMEM((_BPW,), jnp.int32),
        pltpu.VMEM((_CHUNK, _D // 2), jnp.int32),
        pltpu.VMEM((_CHUNK, _D // 2), jnp.int32),
        pltpu.VMEM((_CHUNK, _D // 2), jnp.int32),
        pltpu.VMEM((_CHUNK, _D // 2), jnp.int32),
        pltpu.VMEM((_L,), jnp.float32),
        pltpu.SemaphoreType.DMA,
        pltpu.SemaphoreType.DMA,
        pltpu.SemaphoreType.DMA,
    ],
)(_sc_item_body)


def _sc_user_body(user_hbm, eut_hbm, svi_hbm, svj_hbm,
                  pi_hbm, pj_hbm, reg_hbm,
                  uidx_v, upr_v,
                  ua, via, vja, ub, vib, vjb, pi_v, pj_v, reg_v,
                  sem_a, sem_b):
    c = lax.axis_index("c")
    s = lax.axis_index("s")
    wid = s * _NC + c
    base = wid * _BPW

    pltpu.sync_copy(user_hbm.at[pl.ds(base, _BPW)], uidx_v)

    def prep(i, _):
        sl = pl.ds(i * _L, _L)
        upr_v[sl] = _shuffled_row(uidx_v[sl])
        return 0
    lax.fori_loop(0, _BPW // _L, prep, 0)

    slots = ((ua, via, vja, sem_a), (ub, vib, vjb, sem_b))

    def fire(k, slot):
        u_v, vi_v, vj_v, sem = slot
        sl = pl.ds(k * _CHUNK, _CHUNK)
        ssl = pl.ds(base + k * _CHUNK, _CHUNK)
        return (pltpu.async_copy(eut_hbm.at[upr_v.at[sl]], u_v, sem),
                pltpu.async_copy(svi_hbm.at[ssl], vi_v, sem),
                pltpu.async_copy(svj_hbm.at[ssl], vj_v, sem))

    lane = lax.iota(jnp.int32, _L)
    reg_acc = jnp.zeros((_L,), jnp.float32)
    inflight = fire(0, slots[0])
    for k in range(_NCHUNK):
        u_v, vi_v, vj_v, _ = slots[k % 2]
        for cp in inflight:
            cp.wait()
        if k + 1 < _NCHUNK:
            inflight = fire(k + 1, slots[(k + 1) % 2])

        def group(g, reg_acc, u_v=u_v, vi_v=vi_v, vj_v=vj_v, k=k):
            acc_i = jnp.zeros((_L,), jnp.float32)
            acc_j = jnp.zeros((_L,), jnp.float32)
            for l in range(_L):
                r = g * _L + l
                ss_i = jnp.zeros((_L,), jnp.float32)
                ss_j = jnp.zeros((_L,), jnp.float32)
                for cc in range(2):
                    ulo, uhi = _bf16_pair(u_v, r, cc)
                    vlo, vhi = _bf16_pair(vi_v, r, cc)
                    wlo, whi = _bf16_pair(vj_v, r, cc)
                    ss_i = ss_i + ulo * vlo + uhi * vhi
                    ss_j = ss_j + ulo * wlo + uhi * whi
                    reg_acc = reg_acc + ulo * ulo + uhi * uhi
                acc_i = jnp.where(lane == l, jnp.sum(ss_i), acc_i)
                acc_j = jnp.where(lane == l, jnp.sum(ss_j), acc_j)
            row = pl.ds(k * _CHUNK + g * _L, _L)
            pi_v[row] = acc_i
            pj_v[row] = acc_j
            return reg_acc

        reg_acc = lax.fori_loop(0, _CHUNK // _L, group, reg_acc)

    reg_v[...] = reg_acc
    pltpu.sync_copy(pi_v, pi_hbm.at[pl.ds(base, _BPW)])
    pltpu.sync_copy(pj_v, pj_hbm.at[pl.ds(base, _BPW)])
    pltpu.sync_copy(reg_v, reg_hbm.at[pl.ds(wid * _L, _L)])


_sc_user = functools.partial(
    pl.kernel,
    out_type=(
        jax.ShapeDtypeStruct((_B,), jnp.float32),
        jax.ShapeDtypeStruct((_B,), jnp.float32),
        jax.ShapeDtypeStruct((_NW * _L,), jnp.float32),
    ),
    mesh=_sc_mesh,
    compiler_params=_sc_params,
    scratch_types=[
        pltpu.VMEM((_BPW,), jnp.int32),
        pltpu.VMEM((_BPW,), jnp.int32),
        pltpu.VMEM((_CHUNK, _D // 2), jnp.int32),
        pltpu.VMEM((_CHUNK, _D // 2), jnp.int32),
        pltpu.VMEM((_CHUNK, _D // 2), jnp.int32),
        pltpu.VMEM((_CHUNK, _D // 2), jnp.int32),
        pltpu.VMEM((_CHUNK, _D // 2), jnp.int32),
        pltpu.VMEM((_CHUNK, _D // 2), jnp.int32),
        pltpu.VMEM((_BPW,), jnp.float32),
        pltpu.VMEM((_BPW,), jnp.float32),
        pltpu.VMEM((_L,), jnp.float32),
        pltpu.SemaphoreType.DMA,
        pltpu.SemaphoreType.DMA,
    ],
)(_sc_user_body)


def _bf16_bits(x):
    b = lax.bitcast_convert_type(x, jnp.int32)
    rnd = jnp.int32(0x7FFF) + lax.bitwise_and(
        lax.shift_right_logical(b, 16), 1)
    return lax.shift_right_logical(b + rnd, 16)


def _pair_body(t_ref, o_ref):
    x = jnp.swapaxes(t_ref[...], 0, 1)
    q = x.shape[0] // 32
    x5 = x.reshape(q, 4, 8, _D)
    parts = []
    for b in range(4):
        xb = x5[:, b]
        lo = _bf16_bits(xb[:, :, : _D // 2])
        hi = _bf16_bits(xb[:, :, _D // 2:])
        parts.append(lax.bitwise_or(lo, lax.shift_left(hi, 16)))
    w = jnp.concatenate(parts, axis=-1)
    o_ref[...] = w.reshape(q * 8 * 2 * _D)


def _linear_rows(table):
    n = table.shape[0]
    m = 8 * ((n + 31) // 32)
    blk = 4096
    grid = (m + blk - 1) // blk
    flat = pl.pallas_call(
        _pair_body,
        grid=(grid,),
        in_specs=[pl.BlockSpec((_D, 4 * blk), lambda i: (0, i))],
        out_specs=pl.BlockSpec((blk * 2 * _D,), lambda i: (i,)),
        out_shape=jax.ShapeDtypeStruct((m * 2 * _D,), jnp.int32),
    )(table.T)
    return flat.reshape(4 * m, _D // 2)


def _loss_body(pi_ref, pj_ref, regi_ref, regu_ref, out_ref):
    x = pi_ref[...] - pj_ref[...]
    ls = jnp.minimum(x, 0.0) - jnp.log(1.0 + jnp.exp(-jnp.abs(x)))
    reg = jnp.sum(regi_ref[...]) + jnp.sum(regu_ref[...])
    out_ref[0, 0] = _REG * reg - jnp.sum(ls)


_loss_call = pl.pallas_call(
    _loss_body,
    out_shape=jax.ShapeDtypeStruct((1, 1), jnp.float32),
    out_specs=pl.BlockSpec(memory_space=pltpu.SMEM),
)


def kernel(user, item_i, item_j, embed_user, embed_item):
    ei2 = _linear_rows(embed_item)
    svi, svj, regij = _sc_item(item_i, item_j, ei2)
    eu2 = _linear_rows(embed_user)
    pi, pj, regu = _sc_user(user, eu2, svi, svj)
    loss = _loss_call(pi.reshape(_B // 128, 128),
                      pj.reshape(_B // 128, 128),
                      regij.reshape(_NW * _L // 128, 128),
                      regu.reshape(_NW * _L // 128, 128))[0, 0]
    return (pi, pj, loss)
